# baseline (speedup 1.0000x reference)
import functools

import jax
import jax.numpy as jnp
from jax import lax
from jax.experimental import pallas as pl
from jax.experimental.pallas import tpu as pltpu
from jax.experimental.pallas import tpu_sc as plsc

N = 100000
D = 128
G = 512

CHUNK = 80
N_CHUNKS = N // CHUNK
NW = 32
K_STEPS = -(-N_CHUNKS // NW)
NBUF = 4

_mesh = plsc.VectorSubcoreMesh(core_axis_name="c", subcore_axis_name="s")


@functools.partial(
    pl.kernel,
    out_type=jax.ShapeDtypeStruct((2, G, D), jnp.float32),
    mesh=_mesh,
    scratch_types=[
        pltpu.VMEM((NBUF, CHUNK), jnp.int32),
        pltpu.VMEM((NBUF, CHUNK, D), jnp.float32),
        pltpu.VMEM_SHARED((G, D), jnp.float32),
        pltpu.SemaphoreType.DMA((NBUF,)),
    ],
)
def _sc_segsum(nf_hbm, batch2d_hbm, zeros_hbm, part_hbm,
               idx_v, rows_v, acc, gsem):
    cid = lax.axis_index("c")
    sid = lax.axis_index("s")
    wid = sid * 2 + cid

    pltpu.sync_copy(zeros_hbm.at[pl.ds(sid * 32, 32)], acc.at[pl.ds(sid * 32, 32)])
    plsc.subcore_barrier()

    start_c = 39 * wid + jnp.minimum(wid, 2)
    nch = jnp.where(wid < 2, 40, 39)

    def gather_issue(k, b):
        c = start_c + k
        pltpu.async_copy(batch2d_hbm.at[c], idx_v.at[b], gsem.at[b])
        pltpu.async_copy(nf_hbm.at[pl.ds(c * CHUNK, CHUNK), :], rows_v.at[b],
                         gsem.at[b])

    def gather_wait(b):
        pltpu.make_async_copy(batch2d_hbm.at[0], idx_v.at[b], gsem.at[b]).wait()
        pltpu.make_async_copy(nf_hbm.at[pl.ds(0, CHUNK), :], rows_v.at[b],
                              gsem.at[b]).wait()

    for j in range(NBUF):
        @pl.when(j < nch)
        def _(j=j):
            gather_issue(j, j)

    def body(g, carry):
        for b in range(NBUF):
            k = NBUF * g + b

            @pl.when(k < nch)
            def _():
                gather_wait(b)
                pltpu.sync_copy(rows_v.at[b], acc.at[idx_v.at[b]], add=True)

                j = k + NBUF

                @pl.when(j < nch)
                def _():
                    gather_issue(j, b)

        return carry

    lax.fori_loop(0, K_STEPS // NBUF, body, jnp.int32(0))

    plsc.subcore_barrier()

    pltpu.sync_copy(acc.at[pl.ds(sid * 32, 32)], part_hbm.at[cid, pl.ds(sid * 32, 32)])


def _tc_add(p_ref, o_ref):
    o_ref[...] = p_ref[0] + p_ref[1]


def kernel(node_features, batch):
    zeros = jnp.zeros((G, D), jnp.float32)
    batch2d = batch.reshape(N_CHUNKS, CHUNK)
    partials = _sc_segsum(node_features, batch2d, zeros)
    return pl.pallas_call(
        _tc_add,
        out_shape=jax.ShapeDtypeStruct((G, D), jnp.float32),
    )(partials)


# speedup vs baseline: 1.0278x; 1.0278x over previous
import functools

import jax
import jax.numpy as jnp
from jax import lax
from jax.experimental import pallas as pl
from jax.experimental.pallas import tpu as pltpu
from jax.experimental.pallas import tpu_sc as plsc

N = 100000
D = 128
G = 512

CHUNK = 80
N_CHUNKS = N // CHUNK
NW = 32
K_STEPS = -(-N_CHUNKS // NW)
NBUF = 8
LEAD = NBUF // 2

_mesh = plsc.VectorSubcoreMesh(core_axis_name="c", subcore_axis_name="s")


@functools.partial(
    pl.kernel,
    out_type=jax.ShapeDtypeStruct((2, G, D), jnp.float32),
    mesh=_mesh,
    scratch_types=[
        pltpu.VMEM((NBUF, CHUNK), jnp.int32),
        pltpu.VMEM((NBUF, CHUNK, D), jnp.float32),
        pltpu.VMEM_SHARED((G, D), jnp.float32),
        pltpu.SemaphoreType.DMA((NBUF,)),
        pltpu.SemaphoreType.DMA((NBUF,)),
    ],
)
def _sc_segsum(nf_hbm, batch2d_hbm, zeros_hbm, part_hbm,
               idx_v, rows_v, acc, gsem, ssem):
    cid = lax.axis_index("c")
    sid = lax.axis_index("s")
    wid = sid * 2 + cid

    pltpu.sync_copy(zeros_hbm.at[pl.ds(sid * 32, 32)], acc.at[pl.ds(sid * 32, 32)])

    start_c = 39 * wid + jnp.minimum(wid, 2)
    nch = jnp.where(wid < 2, 40, 39)

    plsc.subcore_barrier()

    def gather_issue(k, b):
        c = start_c + k
        pltpu.async_copy(batch2d_hbm.at[c], idx_v.at[b], gsem.at[b])
        pltpu.async_copy(nf_hbm.at[pl.ds(c * CHUNK, CHUNK), :],
                         rows_v.at[b], gsem.at[b])

    def gather_wait(b):
        pltpu.make_async_copy(batch2d_hbm.at[0], idx_v.at[b], gsem.at[b]).wait()
        pltpu.make_async_copy(nf_hbm.at[pl.ds(0, CHUNK), :], rows_v.at[b],
                              gsem.at[b]).wait()

    def scatter_issue(k, b):
        pltpu.async_copy(rows_v.at[b], acc.at[idx_v.at[b]], ssem.at[b],
                         add=True)

    def scatter_wait(b):
        pltpu.make_async_copy(rows_v.at[b], acc.at[idx_v.at[0]], ssem.at[b]).wait()

    for j in range(LEAD):
        @pl.when(j < nch)
        def _(j=j):
            gather_issue(j, j)

    def body(g, carry):
        for b in range(NBUF):
            k = NBUF * g + b

            @pl.when(k < nch)
            def _():
                gather_wait(b)
                scatter_issue(k, b)

            b2 = (b + LEAD) % NBUF
            m = k - LEAD
            j = k + LEAD

            @pl.when((m >= 0) & (m < nch))
            def _():
                scatter_wait(b2)

            @pl.when(j < nch)
            def _():
                gather_issue(j, b2)

        return carry

    lax.fori_loop(0, K_STEPS // NBUF, body, jnp.int32(0))

    for kk in range(K_STEPS - LEAD, K_STEPS):
        @pl.when(kk < nch)
        def _(kk=kk):
            scatter_wait(kk % NBUF)

    plsc.subcore_barrier()

    pltpu.sync_copy(acc.at[pl.ds(sid * 32, 32)], part_hbm.at[cid, pl.ds(sid * 32, 32)])


def _tc_add(p_ref, o_ref):
    o_ref[...] = p_ref[0] + p_ref[1]


def kernel(node_features, batch):
    zeros = jnp.zeros((G, D), jnp.float32)
    batch2d = batch.reshape(N_CHUNKS, CHUNK)
    partials = _sc_segsum(node_features, batch2d, zeros)
    return pl.pallas_call(
        _tc_add,
        out_shape=jax.ShapeDtypeStruct((G, D), jnp.float32),
    )(partials)


# speedup vs baseline: 1.0713x; 1.0424x over previous
import functools

import jax
import jax.numpy as jnp
from jax import lax
from jax.experimental import pallas as pl
from jax.experimental.pallas import tpu as pltpu
from jax.experimental.pallas import tpu_sc as plsc

N = 100000
D = 128
G = 512

SCAT = 80
CHUNK = 160
N_CHUNKS = N // CHUNK
NW = 32
K_STEPS = -(-N_CHUNKS // NW)
NBUF = 4
LEAD = NBUF // 2
ZR = G // 16

_mesh = plsc.VectorSubcoreMesh(core_axis_name="c", subcore_axis_name="s")


@functools.partial(
    pl.kernel,
    out_type=jax.ShapeDtypeStruct((2, G, D), jnp.float32),
    mesh=_mesh,
    scratch_types=[
        pltpu.VMEM((NBUF, 2, SCAT), jnp.int32),
        pltpu.VMEM((NBUF, CHUNK, D), jnp.float32),
        pltpu.VMEM((ZR, D), jnp.float32),
        pltpu.VMEM_SHARED((G, D), jnp.float32),
        pltpu.SemaphoreType.DMA((NBUF,)),
        pltpu.SemaphoreType.DMA((NBUF,)),
    ],
)
def _sc_segsum(nf_hbm, batch3d_hbm, part_hbm,
               idx_v, rows_v, zbuf, acc, gsem, ssem):
    cid = lax.axis_index("c")
    sid = lax.axis_index("s")
    wid = sid * 2 + cid

    z16 = jnp.zeros((16,), jnp.float32)
    for r in range(ZR):
        for j in range(D // 16):
            zbuf[r, pl.ds(j * 16, 16)] = z16
    pltpu.sync_copy(zbuf, acc.at[pl.ds(sid * ZR, ZR)])

    start_c = 19 * wid + jnp.minimum(wid, 17)
    nch = jnp.where(wid < 17, 20, 19)

    plsc.subcore_barrier()

    def gather_issue(k, b):
        c = start_c + k
        pltpu.async_copy(batch3d_hbm.at[c], idx_v.at[b], gsem.at[b])
        pltpu.async_copy(nf_hbm.at[pl.ds(c * CHUNK, CHUNK), :],
                         rows_v.at[b], gsem.at[b])

    def gather_wait(b):
        pltpu.make_async_copy(batch3d_hbm.at[0], idx_v.at[b], gsem.at[b]).wait()
        pltpu.make_async_copy(nf_hbm.at[pl.ds(0, CHUNK), :], rows_v.at[b],
                              gsem.at[b]).wait()

    def scatter_issue(b):
        for h in range(2):
            pltpu.async_copy(rows_v.at[b, pl.ds(h * SCAT, SCAT), :],
                             acc.at[idx_v.at[b, h]], ssem.at[b], add=True)

    def scatter_wait(b):
        for h in range(2):
            pltpu.make_async_copy(rows_v.at[b, pl.ds(h * SCAT, SCAT), :],
                                  acc.at[idx_v.at[b, 0]], ssem.at[b]).wait()

    for j in range(LEAD):
        @pl.when(j < nch)
        def _(j=j):
            gather_issue(j, j)

    def body(g, carry):
        for b in range(NBUF):
            k = NBUF * g + b

            @pl.when(k < nch)
            def _():
                gather_wait(b)
                scatter_issue(b)

            b2 = (b + LEAD) % NBUF
            m = k - LEAD
            j = k + LEAD

            @pl.when((m >= 0) & (m < nch))
            def _():
                scatter_wait(b2)

            @pl.when(j < nch)
            def _():
                gather_issue(j, b2)

        return carry

    lax.fori_loop(0, K_STEPS // NBUF, body, jnp.int32(0))

    for kk in range(K_STEPS - LEAD, K_STEPS):
        @pl.when(kk < nch)
        def _(kk=kk):
            scatter_wait(kk % NBUF)

    plsc.subcore_barrier()

    pltpu.sync_copy(acc.at[pl.ds(sid * ZR, ZR)], part_hbm.at[cid, pl.ds(sid * ZR, ZR)])


def _tc_add(p_ref, o_ref):
    o_ref[...] = p_ref[0] + p_ref[1]


def kernel(node_features, batch):
    batch3d = batch.reshape(N_CHUNKS, 2, SCAT)
    partials = _sc_segsum(node_features, batch3d)
    return pl.pallas_call(
        _tc_add,
        out_shape=jax.ShapeDtypeStruct((G, D), jnp.float32),
    )(partials)
